# baseline (speedup 1.0000x reference)
import jax
import jax.numpy as jnp
from jax import lax
from jax.experimental import pallas as pl
from jax.experimental.pallas import tpu as pltpu
from jax.experimental.pallas import tpu_sc as plsc

VOCAB = 1000000
EMBED_DIM = 64
BATCH = 4096
SEQ = 200

_NC = 2
_NS = 16
_NW = _NC * _NS
_BB = BATCH // _NW


def _embed_gather(idx_hbm, table2_hbm, out_hbm,
                  idxw, idx2a, idx2b, cola, colb, rawsa, rawsb, outa, outb,
                  sg0, sg1, sw0, sw1):
    wid = lax.axis_index("s") * _NC + lax.axis_index("c")
    idx2 = (idx2a, idx2b)
    coloff = (cola, colb)
    raws = (rawsa, rawsb)
    out_t = (outa, outb)
    sem_g = (sg0, sg1)
    sem_w = (sw0, sw1)

    pltpu.sync_copy(idx_hbm.at[:, pl.ds(wid * _BB, _BB)], idxw)

    iota = lax.iota(jnp.int32, 16)

    def idxprep(s, j):
        for k in range(8):
            v = idxw[s, pl.ds(16 * k, 16)]
            idx2[j][pl.ds(16 * k, 16)] = v >> 1
            coloff[j][pl.ds(16 * k, 16)] = (v & 1) * 64

    def fire_gather(j):
        pltpu.async_copy(table2_hbm.at[idx2[j]], raws[j], sem_g[j])

    def drain_gather(j):
        pltpu.make_async_copy(
            table2_hbm.at[idx2[j]], raws[j], sem_g[j]).wait()

    def fire_wb(j, s):
        pltpu.async_copy(out_t[j], out_hbm.at[s, :, wid], sem_w[j])

    def wait_wb(j, s):
        pltpu.make_async_copy(
            out_t[j], out_hbm.at[s, :, wid], sem_w[j]).wait()

    def transpose(j):
        rowvs = [iota + 16 * k for k in range(8)]
        colvs = [coloff[j][pl.ds(16 * k, 16)] for k in range(8)]

        def f_body(f, carry):
            fi = f >> 3
            fr = f & 7
            for k in range(8):
                val = plsc.load_gather(raws[j], [rowvs[k], colvs[k] + f])
                out_t[j][fi, fr, pl.ds(16 * k, 16)] = val
            return carry

        lax.fori_loop(0, EMBED_DIM, f_body, 0)

    idxprep(0, 0)
    fire_gather(0)

    def step(m, carry):
        for j in range(2):
            s = 2 * m + j
            idxprep(s + 1, 1 - j)
            fire_gather(1 - j)
            drain_gather(j)

            @pl.when(m >= 1)
            def _():
                wait_wb(j, s)

            transpose(j)
            fire_wb(j, s)
        return carry

    lax.fori_loop(0, SEQ // 2 - 1, step, 0)

    s0 = SEQ - 2
    idxprep(SEQ - 1, 1)
    fire_gather(1)
    drain_gather(0)
    wait_wb(0, s0)
    transpose(0)
    fire_wb(0, s0)
    drain_gather(1)
    wait_wb(1, s0 + 1)
    transpose(1)
    fire_wb(1, s0 + 1)
    wait_wb(0, s0)
    wait_wb(1, s0 + 1)


@jax.jit
def kernel(indices, embed_table):
    table2 = embed_table.reshape(VOCAB // 2, 2 * EMBED_DIM)
    idx2d = indices.T.astype(jnp.int32)
    mesh = plsc.VectorSubcoreMesh(core_axis_name="c", subcore_axis_name="s")
    out5 = pl.kernel(
        _embed_gather,
        mesh=mesh,
        out_type=jax.ShapeDtypeStruct(
            (SEQ, EMBED_DIM // 8, BATCH // 128, 8, 128), jnp.float32),
        scratch_types=[
            pltpu.VMEM((SEQ, _BB), jnp.int32),
            pltpu.VMEM((_BB,), jnp.int32),
            pltpu.VMEM((_BB,), jnp.int32),
            pltpu.VMEM((_BB,), jnp.int32),
            pltpu.VMEM((_BB,), jnp.int32),
            pltpu.VMEM((_BB, 2 * EMBED_DIM), jnp.float32),
            pltpu.VMEM((_BB, 2 * EMBED_DIM), jnp.float32),
            pltpu.VMEM((EMBED_DIM // 8, 8, _BB), jnp.float32),
            pltpu.VMEM((EMBED_DIM // 8, 8, _BB), jnp.float32),
            pltpu.SemaphoreType.DMA,
            pltpu.SemaphoreType.DMA,
            pltpu.SemaphoreType.DMA,
            pltpu.SemaphoreType.DMA,
        ],
        compiler_params=pltpu.CompilerParams(
            use_tc_tiling_on_sc=False, needs_layout_passes=False),
    )(idx2d, table2)
    return out5.transpose(2, 4, 0, 1, 3).reshape(BATCH, SEQ, EMBED_DIM)


# speedup vs baseline: 1.7349x; 1.7349x over previous
import jax
import jax.numpy as jnp
from jax import lax
from jax.experimental import pallas as pl
from jax.experimental.pallas import tpu as pltpu
from jax.experimental.pallas import tpu_sc as plsc

VOCAB = 1000000
EMBED_DIM = 64
BATCH = 4096
SEQ = 200

_NC = 2
_NS = 16
_NW = _NC * _NS
_BB = BATCH // _NW


def _embed_gather(idx_hbm, table2_hbm, out_hbm,
                  idxw, idx2a, idx2b, cola, colb, rawsa, rawsb, outa, outb,
                  fvecs, fivecs, f128s, sg0, sg1, sw0, sw1):
    wid = lax.axis_index("s") * _NC + lax.axis_index("c")
    idx2 = (idx2a, idx2b)
    coloff = (cola, colb)
    raws = (rawsa, rawsb)
    out_t = (outa, outb)
    sem_g = (sg0, sg1)
    sem_w = (sw0, sw1)

    pltpu.sync_copy(idx_hbm.at[:, pl.ds(wid * _BB, _BB)], idxw)

    iota = lax.iota(jnp.int32, 16)

    for m in range(4):
        for d in range(16):
            f = 16 * m + ((iota + d) & 15)
            r = 16 * m + d
            fvecs[r, pl.ds(0, 16)] = f
            fivecs[r, pl.ds(0, 16)] = f >> 3
            f128s[r, pl.ds(0, 16)] = (f & 7) * 128

    def idxprep(s, j):
        for k in range(8):
            v = idxw[s, pl.ds(16 * k, 16)]
            idx2[j][pl.ds(16 * k, 16)] = v >> 1
            coloff[j][pl.ds(16 * k, 16)] = (v & 1) * 64

    def fire_gather(j):
        pltpu.async_copy(table2_hbm.at[idx2[j]], raws[j], sem_g[j])

    def drain_gather(j):
        pltpu.make_async_copy(
            table2_hbm.at[idx2[j]], raws[j], sem_g[j]).wait()

    def fire_wb(j, s):
        pltpu.async_copy(out_t[j], out_hbm.at[s, :, wid], sem_w[j])

    def wait_wb(j, s):
        pltpu.make_async_copy(
            out_t[j], out_hbm.at[s, :, wid], sem_w[j]).wait()

    def transpose(j):
        tvs = [iota + 16 * k for k in range(8)]
        cvs = [coloff[j][pl.ds(16 * k, 16)] for k in range(8)]

        def d_body(d, carry):
            for m in range(4):
                r = 16 * m + d
                fv = fvecs[r, pl.ds(0, 16)]
                fiv = fivecs[r, pl.ds(0, 16)]
                f128v = f128s[r, pl.ds(0, 16)]
                for k in range(8):
                    val = plsc.load_gather(raws[j], [tvs[k], cvs[k] + fv])
                    plsc.store_scatter(out_t[j], [fiv, f128v + tvs[k]], val)
            return carry

        lax.fori_loop(0, 16, d_body, 0)

    idxprep(0, 0)
    fire_gather(0)

    def step(m, carry):
        for j in range(2):
            s = 2 * m + j
            idxprep(s + 1, 1 - j)
            fire_gather(1 - j)
            drain_gather(j)

            @pl.when(m >= 1)
            def _():
                wait_wb(j, s)

            transpose(j)
            fire_wb(j, s)
        return carry

    lax.fori_loop(0, SEQ // 2 - 1, step, 0)

    s0 = SEQ - 2
    idxprep(SEQ - 1, 1)
    fire_gather(1)
    drain_gather(0)
    wait_wb(0, s0)
    transpose(0)
    fire_wb(0, s0)
    drain_gather(1)
    wait_wb(1, s0 + 1)
    transpose(1)
    fire_wb(1, s0 + 1)
    wait_wb(0, s0)
    wait_wb(1, s0 + 1)


@jax.jit
def kernel(indices, embed_table):
    table2 = embed_table.reshape(VOCAB // 2, 2 * EMBED_DIM)
    idx2d = indices.T.astype(jnp.int32)
    mesh = plsc.VectorSubcoreMesh(core_axis_name="c", subcore_axis_name="s")
    out5 = pl.kernel(
        _embed_gather,
        mesh=mesh,
        out_type=jax.ShapeDtypeStruct(
            (SEQ, EMBED_DIM // 8, BATCH // 128, 8 * 128), jnp.float32),
        scratch_types=[
            pltpu.VMEM((SEQ, _BB), jnp.int32),
            pltpu.VMEM((_BB,), jnp.int32),
            pltpu.VMEM((_BB,), jnp.int32),
            pltpu.VMEM((_BB,), jnp.int32),
            pltpu.VMEM((_BB,), jnp.int32),
            pltpu.VMEM((_BB, 2 * EMBED_DIM), jnp.float32),
            pltpu.VMEM((_BB, 2 * EMBED_DIM), jnp.float32),
            pltpu.VMEM((EMBED_DIM // 8, 8 * 128), jnp.float32),
            pltpu.VMEM((EMBED_DIM // 8, 8 * 128), jnp.float32),
            pltpu.VMEM((64, 16), jnp.int32),
            pltpu.VMEM((64, 16), jnp.int32),
            pltpu.VMEM((64, 16), jnp.int32),
            pltpu.SemaphoreType.DMA,
            pltpu.SemaphoreType.DMA,
            pltpu.SemaphoreType.DMA,
            pltpu.SemaphoreType.DMA,
        ],
        compiler_params=pltpu.CompilerParams(
            use_tc_tiling_on_sc=False, needs_layout_passes=False),
    )(idx2d, table2)
    out5 = out5.reshape(SEQ, EMBED_DIM // 8, BATCH // 128, 8, 128)
    return out5.transpose(2, 4, 0, 1, 3).reshape(BATCH, SEQ, EMBED_DIM)
